# baseline (speedup 1.0000x reference)
import functools

import jax
import jax.numpy as jnp
from jax import lax
from jax.experimental import pallas as pl
from jax.experimental.pallas import tpu as pltpu
from jax.experimental.pallas import tpu_sc as plsc

TOPK = 32
LANES = 16
NUM_CORES = 2
NUM_SUBCORES = 16
NUM_WORKERS = NUM_CORES * NUM_SUBCORES
HIST8 = 256
HIST4 = 16
CAP = 4096


def _to_ord(xv):
    b = lax.bitcast_convert_type(xv, jnp.int32)
    flip = (b >> 31) | jnp.int32(-2147483648)
    return lax.bitcast_convert_type(b ^ flip, jnp.uint32)


def _popcount_bytes(pm):
    b0 = (pm >> jnp.uint32(24)) & jnp.uint32(1)
    b1 = (pm >> jnp.uint32(16)) & jnp.uint32(1)
    b2 = (pm >> jnp.uint32(8)) & jnp.uint32(1)
    b3 = pm & jnp.uint32(1)
    return b0 + b1 + b2 + b3


def _make_topk_kernel(nrows, ncols):
    rows_per_worker = nrows // NUM_WORKERS
    assert rows_per_worker == 2 and ncols % (8 * LANES) == 0
    nchunks = ncols // LANES
    mesh = plsc.VectorSubcoreMesh(core_axis_name="c", subcore_axis_name="s")

    @functools.partial(
        pl.kernel,
        mesh=mesh,
        compiler_params=pltpu.CompilerParams(needs_layout_passes=False),
        out_type=jax.ShapeDtypeStruct((nrows, ncols), jnp.float32),
        scratch_types=[
            pltpu.VMEM((ncols,), jnp.float32),
            pltpu.VMEM((ncols,), jnp.float32),
            pltpu.VMEM((ncols,), jnp.float32),
            pltpu.VMEM((LANES * HIST8,), jnp.int32),
            pltpu.VMEM((CAP + 144,), jnp.int32),
            pltpu.VMEM((CAP + 144,), jnp.int32),
            pltpu.SemaphoreType.DMA,
            pltpu.SemaphoreType.DMA,
            pltpu.SemaphoreType.DMA,
        ],
    )
    def topk_kernel(x_hbm, out_hbm, row_a, row_b, out_v, hist_v, cand_a,
                    cand_b, sem_a, sem_b, sem_out):
        wid = lax.axis_index("s") * NUM_CORES + lax.axis_index("c")
        lane_iota = lax.iota(jnp.int32, LANES)
        ones_i = jnp.ones((LANES,), jnp.int32)
        zeros_i = jnp.zeros((LANES,), jnp.int32)
        zeros_f = jnp.zeros((LANES,), jnp.float32)

        def top2_scan(row_v):
            zu = jnp.zeros((LANES,), jnp.uint32)

            def body(o, carry):
                m1s = list(carry[:8])
                m2s = list(carry[8:])
                for s in range(8):
                    u = _to_ord(row_v[pl.ds((o * 8 + s) * LANES, LANES)])
                    m2s[s] = jnp.maximum(m2s[s], jnp.minimum(m1s[s], u))
                    m1s[s] = jnp.maximum(m1s[s], u)
                return tuple(m1s + m2s)
            carry = lax.fori_loop(0, nchunks // 64, body, (zu,) * 16)

            pairs = list(zip(carry[:8], carry[8:]))
            while len(pairs) > 1:
                nxt = []
                for (a1, a2), (b1, b2) in zip(pairs[::2], pairs[1::2]):
                    hi = jnp.maximum(a1, b1)
                    lo = jnp.maximum(jnp.minimum(a1, b1),
                                     jnp.maximum(a2, b2))
                    nxt.append((hi, lo))
                pairs = nxt
            _, m2 = pairs[0]
            return jnp.min(m2)

        def compact(row_v, cand_i, thresh):
            th_v = jnp.broadcast_to(thresh, (LANES,))

            def cb(o, ptr):
                base = jnp.minimum(ptr, jnp.int32(CAP))
                masks, cnts = [], []
                for s in range(8):
                    u = _to_ord(row_v[pl.ds((o * 8 + s) * LANES, LANES)])
                    matc = u >= th_v
                    masks.append(matc)
                    cnts.append(jnp.sum(matc.astype(jnp.int32)))
                offs = [base]
                for s in range(8):
                    offs.append(offs[s] + cnts[s])
                for s in range(8):
                    plsc.store_compressed(
                        cand_i.at[pl.ds(offs[s], LANES)],
                        lane_iota + (o * 8 + s) * LANES, mask=masks[s])
                return ptr + (offs[8] - base)
            return lax.fori_loop(0, nchunks // 8, cb, jnp.int32(0))

        def find_top(krem, nbuckets):
            def wcond(carry):
                _, cum, _ = carry
                return cum < krem

            def wbody(carry):
                c, cum, _ = carry
                v = hist_v[pl.ds(c * LANES, LANES)]
                hist_v[pl.ds(c * LANES, LANES)] = zeros_i
                return c - 1, cum + jnp.sum(v), cum
            c, cum, prev = lax.while_loop(
                wcond, wbody,
                (jnp.int32(nbuckets - 1), jnp.int32(0), jnp.int32(0)))
            d = c + 1

            def zb(b, _):
                hist_v[pl.ds(b * LANES, LANES)] = zeros_i
                return 0
            lax.fori_loop(0, d, zb, 0)
            return d, prev, cum - prev

        def full_round(row_v, shift, pm, pb, krem, masked):
            def hb(o, _):
                for s in range(8):
                    u = _to_ord(row_v[pl.ds((o * 8 + s) * LANES, LANES)])
                    digit = ((u >> shift) & jnp.uint32(0xFF)).astype(jnp.int32)
                    idx = digit * LANES + lane_iota
                    if masked:
                        matc = (u & pm) == pb
                        plsc.addupdate_scatter(hist_v, [idx], ones_i,
                                               mask=matc)
                    else:
                        plsc.addupdate_scatter(hist_v, [idx], ones_i)
                return 0
            lax.fori_loop(0, nchunks // 8, hb, 0)

            d, ca, ceq = find_top(krem, HIST8)
            pb = pb | (d.astype(jnp.uint32) << shift)
            pm = pm | (jnp.uint32(0xFF) << shift)
            krem = krem - ca
            return pm, pb, krem, ceq

        def select_row(row_v, cand_i):
            T = top2_scan(row_v)
            c_t = compact(row_v, cand_i, T)

            def fast(_):
                return (jnp.uint32(0), jnp.uint32(0), jnp.int32(TOPK), c_t,
                        jnp.int32(8))

            def slow(_):
                pm, pb, krem, ceq = full_round(
                    row_v, jnp.uint32(24), jnp.uint32(0), jnp.uint32(0),
                    jnp.int32(TOPK), masked=False)

                def esc_body(rnd, carry):
                    pm, pb, krem, ceq = carry

                    def run(_):
                        shift = (jnp.uint32(24)
                                 - jnp.uint32(8) * rnd.astype(jnp.uint32))
                        return full_round(row_v, shift, pm, pb, krem,
                                          masked=True)
                    return lax.cond(
                        (TOPK - krem) + ceq > CAP, run,
                        lambda _: (pm, pb, krem, ceq), 0)
                pm, pb, krem, ceq = lax.fori_loop(
                    1, 4, esc_body, (pm, pb, krem, ceq))

                compact(row_v, cand_i, pb)
                cnt = (TOPK - krem) + ceq
                nrounds = ((jnp.uint32(4) - _popcount_bytes(pm))
                           * jnp.uint32(2)).astype(jnp.int32)
                return pm, pb, krem, cnt, nrounds

            pm, pb, krem, cnt_c, nrounds = lax.cond(
                c_t <= CAP, fast, slow, 0)
            cchunks = (cnt_c + LANES - 1) // LANES
            cnt_v = jnp.broadcast_to(cnt_c, (LANES,))
            nbits = jnp.uint32(4) * nrounds.astype(jnp.uint32)

            def cr_body(i, carry):
                pm2, pb2, krem2 = carry
                shift = nbits - jnp.uint32(4) * (i.astype(jnp.uint32)
                                                 + jnp.uint32(1))

                def chb(j, _):
                    idx = cand_i[pl.ds(j * LANES, LANES)]
                    valid = (j * LANES + lane_iota) < cnt_v
                    xg = plsc.load_gather(row_v, [idx], mask=valid)
                    u = _to_ord(xg)
                    matc = jnp.logical_and(valid, (u & pm2) == pb2)
                    digit = ((u >> shift) & jnp.uint32(0xF)).astype(jnp.int32)
                    plsc.addupdate_scatter(
                        hist_v, [digit * LANES + lane_iota], ones_i,
                        mask=matc)
                    return 0
                lax.fori_loop(0, cchunks, chb, 0)

                d, excl, _ = find_top(krem2, HIST4)
                pb2 = pb2 | (d.astype(jnp.uint32) << shift)
                pm2 = pm2 | (jnp.uint32(0xF) << shift)
                krem2 = krem2 - excl
                return pm2, pb2, krem2

            _, t, m = lax.fori_loop(0, nrounds, cr_body, (pm, pb, krem))
            return t, m, cnt_c, nrounds == 0

        def emit_row(row_v, cand_i, st):
            t, m, cnt_c, allfull = st
            t_v = jnp.broadcast_to(t, (LANES,))
            m_v = jnp.broadcast_to(m, (LANES,))
            cnt_v = jnp.broadcast_to(cnt_c, (LANES,))
            cchunks = (cnt_c + LANES - 1) // LANES

            def emit_scatter(_):
                def sb(i, eqrun):
                    idx = cand_i[pl.ds(i * LANES, LANES)]
                    valid = (i * LANES + lane_iota) < cnt_v
                    xg = plsc.load_gather(row_v, [idx], mask=valid)
                    u = _to_ord(xg)
                    gt = jnp.logical_and(u > t_v, valid)
                    eq = jnp.logical_and(u == t_v, valid)
                    cs = jnp.cumsum(eq.astype(jnp.int32))
                    keep = jnp.logical_or(
                        gt, jnp.logical_and(eq, (cs + eqrun) <= m_v))
                    plsc.store_scatter(out_v, [idx], xg, mask=keep)
                    return eqrun + jnp.max(cs)
                lax.fori_loop(0, cchunks, sb, jnp.int32(0))
                return 0

            def emit_scan(_):
                def ob(i, eqrun):
                    xv = row_v[pl.ds(i * LANES, LANES)]
                    u = _to_ord(xv)
                    gt = u > t_v
                    eq = u == t_v
                    cs = jnp.cumsum(eq.astype(jnp.int32))
                    keep = jnp.logical_or(
                        gt, jnp.logical_and(eq, (cs + eqrun) <= m_v))
                    out_v[pl.ds(i * LANES, LANES)] = jnp.where(
                        keep, xv, zeros_f)
                    return eqrun + jnp.max(cs)
                lax.fori_loop(0, nchunks, ob, jnp.int32(0))
                return 0

            lax.cond(allfull, emit_scan, emit_scatter, 0)

        def restore_row(cand_i, st):
            _, _, cnt_c, allfull = st
            cnt_v = jnp.broadcast_to(cnt_c, (LANES,))
            cchunks = (cnt_c + LANES - 1) // LANES

            def restore_scatter(_):
                def rb(i, _):
                    idx = cand_i[pl.ds(i * LANES, LANES)]
                    valid = (i * LANES + lane_iota) < cnt_v
                    plsc.store_scatter(out_v, [idx], zeros_f, mask=valid)
                    return 0
                lax.fori_loop(0, cchunks, rb, 0)
                return 0

            def restore_all(_):
                def zb(i, _):
                    out_v[pl.ds(i * LANES, LANES)] = zeros_f
                    return 0
                lax.fori_loop(0, nchunks, zb, 0)
                return 0

            lax.cond(allfull, restore_all, restore_scatter, 0)

        r0 = wid * rows_per_worker
        cp_a = pltpu.async_copy(x_hbm.at[r0], row_a, sem_a)
        cp_b = pltpu.async_copy(x_hbm.at[r0 + 1], row_b, sem_b)

        def zout(i, _):
            out_v[pl.ds(i * LANES, LANES)] = zeros_f
            return 0
        lax.fori_loop(0, nchunks, zout, 0)

        def zhist(i, _):
            hist_v[pl.ds(i * LANES, LANES)] = zeros_i
            return 0
        lax.fori_loop(0, (LANES * HIST8) // LANES, zhist, 0)

        def zcand(i, _):
            cand_a[pl.ds(i * LANES, LANES)] = zeros_i
            cand_b[pl.ds(i * LANES, LANES)] = zeros_i
            return 0
        lax.fori_loop(0, (CAP + 144) // LANES, zcand, 0)

        cp_a.wait()
        st_a = select_row(row_a, cand_a)
        emit_row(row_a, cand_a, st_a)
        out_dma = pltpu.async_copy(out_v, out_hbm.at[r0], sem_out)
        cp_b.wait()
        st_b = select_row(row_b, cand_b)
        out_dma.wait()
        restore_row(cand_a, st_a)
        emit_row(row_b, cand_b, st_b)
        pltpu.sync_copy(out_v, out_hbm.at[r0 + 1])

    return topk_kernel


@jax.jit
def kernel(x):
    nrows, ncols = x.shape
    return _make_topk_kernel(nrows, ncols)(x)


# speedup vs baseline: 1.1056x; 1.1056x over previous
import functools

import jax
import jax.numpy as jnp
from jax import lax
from jax.experimental import pallas as pl
from jax.experimental.pallas import tpu as pltpu
from jax.experimental.pallas import tpu_sc as plsc

TOPK = 32
LANES = 16
NUM_CORES = 2
NUM_SUBCORES = 16
NUM_WORKERS = NUM_CORES * NUM_SUBCORES
HIST8 = 256
HIST4 = 16
CAP = 4096


def _to_ord(xv):
    b = lax.bitcast_convert_type(xv, jnp.int32)
    flip = (b >> 31) | jnp.int32(-2147483648)
    return lax.bitcast_convert_type(b ^ flip, jnp.uint32)


def _from_ord(u):
    ui = lax.bitcast_convert_type(u, jnp.int32)
    flip = ((~ui) >> 31) | jnp.int32(-2147483648)
    return lax.bitcast_convert_type(ui ^ flip, jnp.float32)


def _popcount_bytes(pm):
    b0 = (pm >> jnp.uint32(24)) & jnp.uint32(1)
    b1 = (pm >> jnp.uint32(16)) & jnp.uint32(1)
    b2 = (pm >> jnp.uint32(8)) & jnp.uint32(1)
    b3 = pm & jnp.uint32(1)
    return b0 + b1 + b2 + b3


def _make_topk_kernel(nrows, ncols):
    rows_per_worker = nrows // NUM_WORKERS
    assert rows_per_worker == 2 and ncols % (8 * LANES) == 0
    nchunks = ncols // LANES
    mesh = plsc.VectorSubcoreMesh(core_axis_name="c", subcore_axis_name="s")

    @functools.partial(
        pl.kernel,
        mesh=mesh,
        compiler_params=pltpu.CompilerParams(needs_layout_passes=False),
        out_type=jax.ShapeDtypeStruct((nrows, ncols), jnp.float32),
        scratch_types=[
            pltpu.VMEM((ncols,), jnp.float32),
            pltpu.VMEM((ncols,), jnp.float32),
            pltpu.VMEM((ncols,), jnp.float32),
            pltpu.VMEM((LANES * HIST8,), jnp.int32),
            pltpu.VMEM((CAP + 144,), jnp.int32),
            pltpu.VMEM((CAP + 144,), jnp.int32),
            pltpu.VMEM((CAP + 144,), jnp.uint32),
            pltpu.SemaphoreType.DMA,
            pltpu.SemaphoreType.DMA,
            pltpu.SemaphoreType.DMA,
        ],
    )
    def topk_kernel(x_hbm, out_hbm, row_a, row_b, out_v, hist_v, cand_a,
                    cand_b, cand_u, sem_a, sem_b, sem_out):
        wid = lax.axis_index("s") * NUM_CORES + lax.axis_index("c")
        lane_iota = lax.iota(jnp.int32, LANES)
        ones_i = jnp.ones((LANES,), jnp.int32)
        zeros_i = jnp.zeros((LANES,), jnp.int32)
        zeros_f = jnp.zeros((LANES,), jnp.float32)

        def top2_scan(row_v):
            zu = jnp.zeros((LANES,), jnp.uint32)

            def body(o, carry):
                m1s = list(carry[:8])
                m2s = list(carry[8:])
                for s in range(8):
                    u = _to_ord(row_v[pl.ds((o * 8 + s) * LANES, LANES)])
                    m2s[s] = jnp.maximum(m2s[s], jnp.minimum(m1s[s], u))
                    m1s[s] = jnp.maximum(m1s[s], u)
                return tuple(m1s + m2s)
            carry = lax.fori_loop(0, nchunks // 64, body, (zu,) * 16)

            pairs = list(zip(carry[:8], carry[8:]))
            while len(pairs) > 1:
                nxt = []
                for (a1, a2), (b1, b2) in zip(pairs[::2], pairs[1::2]):
                    hi = jnp.maximum(a1, b1)
                    lo = jnp.maximum(jnp.minimum(a1, b1),
                                     jnp.maximum(a2, b2))
                    nxt.append((hi, lo))
                pairs = nxt
            _, m2 = pairs[0]
            return jnp.min(m2)

        def compact(row_v, cand_i, thresh):
            th_v = jnp.broadcast_to(thresh, (LANES,))

            def cb(o, ptr):
                base = jnp.minimum(ptr, jnp.int32(CAP))
                masks, cnts = [], []
                for s in range(8):
                    u = _to_ord(row_v[pl.ds((o * 8 + s) * LANES, LANES)])
                    matc = u >= th_v
                    masks.append(matc)
                    cnts.append(jnp.sum(matc.astype(jnp.int32)))
                offs = [base]
                for s in range(8):
                    offs.append(offs[s] + cnts[s])
                for s in range(8):
                    plsc.store_compressed(
                        cand_i.at[pl.ds(offs[s], LANES)],
                        lane_iota + (o * 8 + s) * LANES, mask=masks[s])
                return ptr + (offs[8] - base)
            return lax.fori_loop(0, nchunks // 8, cb, jnp.int32(0))

        def find_top(krem, nbuckets):
            def wcond(carry):
                _, cum, _ = carry
                return cum < krem

            def wbody(carry):
                c, cum, _ = carry
                v = hist_v[pl.ds(c * LANES, LANES)]
                hist_v[pl.ds(c * LANES, LANES)] = zeros_i
                return c - 1, cum + jnp.sum(v), cum
            c, cum, prev = lax.while_loop(
                wcond, wbody,
                (jnp.int32(nbuckets - 1), jnp.int32(0), jnp.int32(0)))
            d = c + 1

            def zb(b, _):
                hist_v[pl.ds(b * LANES, LANES)] = zeros_i
                return 0
            lax.fori_loop(0, d, zb, 0)
            return d, prev, cum - prev

        def full_round(row_v, shift, pm, pb, krem, masked):
            def hb(o, _):
                for s in range(8):
                    u = _to_ord(row_v[pl.ds((o * 8 + s) * LANES, LANES)])
                    digit = ((u >> shift) & jnp.uint32(0xFF)).astype(jnp.int32)
                    idx = digit * LANES + lane_iota
                    if masked:
                        matc = (u & pm) == pb
                        plsc.addupdate_scatter(hist_v, [idx], ones_i,
                                               mask=matc)
                    else:
                        plsc.addupdate_scatter(hist_v, [idx], ones_i)
                return 0
            lax.fori_loop(0, nchunks // 8, hb, 0)

            d, ca, ceq = find_top(krem, HIST8)
            pb = pb | (d.astype(jnp.uint32) << shift)
            pm = pm | (jnp.uint32(0xFF) << shift)
            krem = krem - ca
            return pm, pb, krem, ceq

        def select_row(row_v, cand_i):
            T = top2_scan(row_v)
            c_t = compact(row_v, cand_i, T)

            def fast(_):
                return (jnp.uint32(0), jnp.uint32(0), jnp.int32(TOPK), c_t,
                        jnp.int32(8))

            def slow(_):
                pm, pb, krem, ceq = full_round(
                    row_v, jnp.uint32(24), jnp.uint32(0), jnp.uint32(0),
                    jnp.int32(TOPK), masked=False)

                def esc_body(rnd, carry):
                    pm, pb, krem, ceq = carry

                    def run(_):
                        shift = (jnp.uint32(24)
                                 - jnp.uint32(8) * rnd.astype(jnp.uint32))
                        return full_round(row_v, shift, pm, pb, krem,
                                          masked=True)
                    return lax.cond(
                        (TOPK - krem) + ceq > CAP, run,
                        lambda _: (pm, pb, krem, ceq), 0)
                pm, pb, krem, ceq = lax.fori_loop(
                    1, 4, esc_body, (pm, pb, krem, ceq))

                compact(row_v, cand_i, pb)
                cnt = (TOPK - krem) + ceq
                nrounds = ((jnp.uint32(4) - _popcount_bytes(pm))
                           * jnp.uint32(2)).astype(jnp.int32)
                return pm, pb, krem, cnt, nrounds

            pm, pb, krem, cnt_c, nrounds = lax.cond(
                c_t <= CAP, fast, slow, 0)
            cchunks = (cnt_c + LANES - 1) // LANES
            cnt_v = jnp.broadcast_to(cnt_c, (LANES,))
            nbits = jnp.uint32(4) * nrounds.astype(jnp.uint32)

            def gb(j, _):
                idx = cand_i[pl.ds(j * LANES, LANES)]
                valid = (j * LANES + lane_iota) < cnt_v
                xg = plsc.load_gather(row_v, [idx], mask=valid)
                cand_u[pl.ds(j * LANES, LANES)] = _to_ord(xg)
                return 0
            lax.fori_loop(0, cchunks, gb, 0)

            def cr_body(i, carry):
                pm2, pb2, krem2 = carry
                shift = nbits - jnp.uint32(4) * (i.astype(jnp.uint32)
                                                 + jnp.uint32(1))

                def chb(j, _):
                    u = cand_u[pl.ds(j * LANES, LANES)]
                    valid = (j * LANES + lane_iota) < cnt_v
                    matc = jnp.logical_and(valid, (u & pm2) == pb2)
                    digit = ((u >> shift) & jnp.uint32(0xF)).astype(jnp.int32)
                    plsc.addupdate_scatter(
                        hist_v, [digit * LANES + lane_iota], ones_i,
                        mask=matc)
                    return 0
                lax.fori_loop(0, cchunks, chb, 0)

                d, excl, _ = find_top(krem2, HIST4)
                pb2 = pb2 | (d.astype(jnp.uint32) << shift)
                pm2 = pm2 | (jnp.uint32(0xF) << shift)
                krem2 = krem2 - excl
                return pm2, pb2, krem2

            _, t, m = lax.fori_loop(0, nrounds, cr_body, (pm, pb, krem))
            return t, m, cnt_c, nrounds == 0

        def emit_row(row_v, cand_i, st):
            t, m, cnt_c, allfull = st
            t_v = jnp.broadcast_to(t, (LANES,))
            m_v = jnp.broadcast_to(m, (LANES,))
            cnt_v = jnp.broadcast_to(cnt_c, (LANES,))
            cchunks = (cnt_c + LANES - 1) // LANES

            def emit_scatter(_):
                def sb(i, eqrun):
                    idx = cand_i[pl.ds(i * LANES, LANES)]
                    valid = (i * LANES + lane_iota) < cnt_v
                    u = cand_u[pl.ds(i * LANES, LANES)]
                    xg = _from_ord(u)
                    gt = jnp.logical_and(u > t_v, valid)
                    eq = jnp.logical_and(u == t_v, valid)
                    cs = jnp.cumsum(eq.astype(jnp.int32))
                    keep = jnp.logical_or(
                        gt, jnp.logical_and(eq, (cs + eqrun) <= m_v))
                    plsc.store_scatter(out_v, [idx], xg, mask=keep)
                    return eqrun + jnp.max(cs)
                lax.fori_loop(0, cchunks, sb, jnp.int32(0))
                return 0

            def emit_scan(_):
                def ob(i, eqrun):
                    xv = row_v[pl.ds(i * LANES, LANES)]
                    u = _to_ord(xv)
                    gt = u > t_v
                    eq = u == t_v
                    cs = jnp.cumsum(eq.astype(jnp.int32))
                    keep = jnp.logical_or(
                        gt, jnp.logical_and(eq, (cs + eqrun) <= m_v))
                    out_v[pl.ds(i * LANES, LANES)] = jnp.where(
                        keep, xv, zeros_f)
                    return eqrun + jnp.max(cs)
                lax.fori_loop(0, nchunks, ob, jnp.int32(0))
                return 0

            lax.cond(allfull, emit_scan, emit_scatter, 0)

        def restore_row(cand_i, st):
            _, _, cnt_c, allfull = st
            cnt_v = jnp.broadcast_to(cnt_c, (LANES,))
            cchunks = (cnt_c + LANES - 1) // LANES

            def restore_scatter(_):
                def rb(i, _):
                    idx = cand_i[pl.ds(i * LANES, LANES)]
                    valid = (i * LANES + lane_iota) < cnt_v
                    plsc.store_scatter(out_v, [idx], zeros_f, mask=valid)
                    return 0
                lax.fori_loop(0, cchunks, rb, 0)
                return 0

            def restore_all(_):
                def zb(i, _):
                    out_v[pl.ds(i * LANES, LANES)] = zeros_f
                    return 0
                lax.fori_loop(0, nchunks, zb, 0)
                return 0

            lax.cond(allfull, restore_all, restore_scatter, 0)

        r0 = wid * rows_per_worker
        cp_a = pltpu.async_copy(x_hbm.at[r0], row_a, sem_a)
        cp_b = pltpu.async_copy(x_hbm.at[r0 + 1], row_b, sem_b)

        def zout(i, _):
            out_v[pl.ds(i * LANES, LANES)] = zeros_f
            return 0
        lax.fori_loop(0, nchunks, zout, 0)

        def zhist(i, _):
            hist_v[pl.ds(i * LANES, LANES)] = zeros_i
            return 0
        lax.fori_loop(0, (LANES * HIST8) // LANES, zhist, 0)

        def zcand(i, _):
            cand_a[pl.ds(i * LANES, LANES)] = zeros_i
            cand_b[pl.ds(i * LANES, LANES)] = zeros_i
            return 0
        lax.fori_loop(0, (CAP + 144) // LANES, zcand, 0)

        cp_a.wait()
        st_a = select_row(row_a, cand_a)
        emit_row(row_a, cand_a, st_a)
        out_dma = pltpu.async_copy(out_v, out_hbm.at[r0], sem_out)
        cp_b.wait()
        st_b = select_row(row_b, cand_b)
        out_dma.wait()
        restore_row(cand_a, st_a)
        emit_row(row_b, cand_b, st_b)
        pltpu.sync_copy(out_v, out_hbm.at[r0 + 1])

    return topk_kernel


@jax.jit
def kernel(x):
    nrows, ncols = x.shape
    return _make_topk_kernel(nrows, ncols)(x)


# speedup vs baseline: 1.2431x; 1.1244x over previous
import functools

import jax
import jax.numpy as jnp
from jax import lax
from jax.experimental import pallas as pl
from jax.experimental.pallas import tpu as pltpu
from jax.experimental.pallas import tpu_sc as plsc

TOPK = 32
LANES = 16
NUM_CORES = 2
NUM_SUBCORES = 16
NUM_WORKERS = NUM_CORES * NUM_SUBCORES
HIST8 = 256
HIST4 = 16
CAP = 4096


def _to_ord(xv):
    b = lax.bitcast_convert_type(xv, jnp.int32)
    flip = (b >> 31) | jnp.int32(-2147483648)
    return lax.bitcast_convert_type(b ^ flip, jnp.uint32)


def _from_ord(u):
    ui = lax.bitcast_convert_type(u, jnp.int32)
    flip = ((~ui) >> 31) | jnp.int32(-2147483648)
    return lax.bitcast_convert_type(ui ^ flip, jnp.float32)


def _popcount_bytes(pm):
    b0 = (pm >> jnp.uint32(24)) & jnp.uint32(1)
    b1 = (pm >> jnp.uint32(16)) & jnp.uint32(1)
    b2 = (pm >> jnp.uint32(8)) & jnp.uint32(1)
    b3 = pm & jnp.uint32(1)
    return b0 + b1 + b2 + b3


def _make_topk_kernel(nrows, ncols):
    rows_per_worker = nrows // NUM_WORKERS
    assert rows_per_worker == 2 and ncols % (8 * LANES) == 0
    nchunks = ncols // LANES
    mesh = plsc.VectorSubcoreMesh(core_axis_name="c", subcore_axis_name="s")

    @functools.partial(
        pl.kernel,
        mesh=mesh,
        compiler_params=pltpu.CompilerParams(needs_layout_passes=False),
        out_type=jax.ShapeDtypeStruct((nrows, ncols), jnp.float32),
        scratch_types=[
            pltpu.VMEM((ncols,), jnp.float32),
            pltpu.VMEM((ncols,), jnp.float32),
            pltpu.VMEM((ncols,), jnp.float32),
            pltpu.VMEM((LANES * HIST8,), jnp.int32),
            pltpu.VMEM((CAP + 144,), jnp.int32),
            pltpu.VMEM((CAP + 144,), jnp.int32),
            pltpu.VMEM((CAP + 144,), jnp.uint32),
            pltpu.SemaphoreType.DMA,
            pltpu.SemaphoreType.DMA,
            pltpu.SemaphoreType.DMA,
        ],
    )
    def topk_kernel(x_hbm, out_hbm, row_a, row_b, out_v, hist_v, cand_a,
                    cand_b, cand_u, sem_a, sem_b, sem_out):
        wid = lax.axis_index("s") * NUM_CORES + lax.axis_index("c")
        lane_iota = lax.iota(jnp.int32, LANES)
        ones_i = jnp.ones((LANES,), jnp.int32)
        zeros_i = jnp.zeros((LANES,), jnp.int32)
        zeros_f = jnp.zeros((LANES,), jnp.float32)

        def top2_scan(row_v):
            zu = jnp.zeros((LANES,), jnp.uint32)

            def body(o, carry):
                m1s = list(carry[:8])
                m2s = list(carry[8:])
                for s in range(8):
                    u = _to_ord(row_v[pl.ds((o * 8 + s) * LANES, LANES)])
                    m2s[s] = jnp.maximum(m2s[s], jnp.minimum(m1s[s], u))
                    m1s[s] = jnp.maximum(m1s[s], u)
                return tuple(m1s + m2s)
            carry = lax.fori_loop(0, nchunks // 8, body, (zu,) * 16)

            pairs = list(zip(carry[:8], carry[8:]))
            while len(pairs) > 1:
                nxt = []
                for (a1, a2), (b1, b2) in zip(pairs[::2], pairs[1::2]):
                    hi = jnp.maximum(a1, b1)
                    lo = jnp.maximum(jnp.minimum(a1, b1),
                                     jnp.maximum(a2, b2))
                    nxt.append((hi, lo))
                pairs = nxt
            _, m2 = pairs[0]
            return jnp.min(m2)

        def compact(row_v, cand_i, thresh):
            th_v = jnp.broadcast_to(thresh, (LANES,))

            def cb(o, ptr):
                base = jnp.minimum(ptr, jnp.int32(CAP))
                masks, cnts = [], []
                for s in range(8):
                    u = _to_ord(row_v[pl.ds((o * 8 + s) * LANES, LANES)])
                    matc = u >= th_v
                    masks.append(matc)
                    cnts.append(jnp.sum(matc.astype(jnp.int32)))
                offs = [base]
                for s in range(8):
                    offs.append(offs[s] + cnts[s])
                for s in range(8):
                    plsc.store_compressed(
                        cand_i.at[pl.ds(offs[s], LANES)],
                        lane_iota + (o * 8 + s) * LANES, mask=masks[s])
                return ptr + (offs[8] - base)
            return lax.fori_loop(0, nchunks // 8, cb, jnp.int32(0))

        def find_top(krem, nbuckets):
            def wcond(carry):
                _, cum, _ = carry
                return cum < krem

            def wbody(carry):
                c, cum, _ = carry
                v = hist_v[pl.ds(c * LANES, LANES)]
                hist_v[pl.ds(c * LANES, LANES)] = zeros_i
                return c - 1, cum + jnp.sum(v), cum
            c, cum, prev = lax.while_loop(
                wcond, wbody,
                (jnp.int32(nbuckets - 1), jnp.int32(0), jnp.int32(0)))
            d = c + 1

            def zb(b, _):
                hist_v[pl.ds(b * LANES, LANES)] = zeros_i
                return 0
            lax.fori_loop(0, d, zb, 0)
            return d, prev, cum - prev

        def full_round(row_v, shift, pm, pb, krem, masked):
            def hb(o, _):
                for s in range(8):
                    u = _to_ord(row_v[pl.ds((o * 8 + s) * LANES, LANES)])
                    digit = ((u >> shift) & jnp.uint32(0xFF)).astype(jnp.int32)
                    idx = digit * LANES + lane_iota
                    if masked:
                        matc = (u & pm) == pb
                        plsc.addupdate_scatter(hist_v, [idx], ones_i,
                                               mask=matc)
                    else:
                        plsc.addupdate_scatter(hist_v, [idx], ones_i)
                return 0
            lax.fori_loop(0, nchunks // 8, hb, 0)

            d, ca, ceq = find_top(krem, HIST8)
            pb = pb | (d.astype(jnp.uint32) << shift)
            pm = pm | (jnp.uint32(0xFF) << shift)
            krem = krem - ca
            return pm, pb, krem, ceq

        def select_row(row_v, cand_i):
            T = top2_scan(row_v)
            c_t = compact(row_v, cand_i, T)

            def fast(_):
                return (jnp.uint32(0), jnp.uint32(0), jnp.int32(TOPK), c_t,
                        jnp.int32(8))

            def slow(_):
                pm, pb, krem, ceq = full_round(
                    row_v, jnp.uint32(24), jnp.uint32(0), jnp.uint32(0),
                    jnp.int32(TOPK), masked=False)

                def esc_body(rnd, carry):
                    pm, pb, krem, ceq = carry

                    def run(_):
                        shift = (jnp.uint32(24)
                                 - jnp.uint32(8) * rnd.astype(jnp.uint32))
                        return full_round(row_v, shift, pm, pb, krem,
                                          masked=True)
                    return lax.cond(
                        (TOPK - krem) + ceq > CAP, run,
                        lambda _: (pm, pb, krem, ceq), 0)
                pm, pb, krem, ceq = lax.fori_loop(
                    1, 4, esc_body, (pm, pb, krem, ceq))

                compact(row_v, cand_i, pb)
                cnt = (TOPK - krem) + ceq
                nrounds = ((jnp.uint32(4) - _popcount_bytes(pm))
                           * jnp.uint32(2)).astype(jnp.int32)
                return pm, pb, krem, cnt, nrounds

            pm, pb, krem, cnt_c, nrounds = lax.cond(
                c_t <= CAP, fast, slow, 0)
            cchunks = (cnt_c + LANES - 1) // LANES
            cnt_v = jnp.broadcast_to(cnt_c, (LANES,))
            nbits = jnp.uint32(4) * nrounds.astype(jnp.uint32)

            def gb(j, _):
                idx = cand_i[pl.ds(j * LANES, LANES)]
                valid = (j * LANES + lane_iota) < cnt_v
                xg = plsc.load_gather(row_v, [idx], mask=valid)
                cand_u[pl.ds(j * LANES, LANES)] = _to_ord(xg)
                return 0
            lax.fori_loop(0, cchunks, gb, 0)

            def cr_body(i, carry):
                pm2, pb2, krem2 = carry
                shift = nbits - jnp.uint32(4) * (i.astype(jnp.uint32)
                                                 + jnp.uint32(1))

                def chb(j, _):
                    u = cand_u[pl.ds(j * LANES, LANES)]
                    valid = (j * LANES + lane_iota) < cnt_v
                    matc = jnp.logical_and(valid, (u & pm2) == pb2)
                    digit = ((u >> shift) & jnp.uint32(0xF)).astype(jnp.int32)
                    plsc.addupdate_scatter(
                        hist_v, [digit * LANES + lane_iota], ones_i,
                        mask=matc)
                    return 0
                lax.fori_loop(0, cchunks, chb, 0)

                d, excl, _ = find_top(krem2, HIST4)
                pb2 = pb2 | (d.astype(jnp.uint32) << shift)
                pm2 = pm2 | (jnp.uint32(0xF) << shift)
                krem2 = krem2 - excl
                return pm2, pb2, krem2

            _, t, m = lax.fori_loop(0, nrounds, cr_body, (pm, pb, krem))
            return t, m, cnt_c, nrounds == 0

        def emit_row(row_v, cand_i, st):
            t, m, cnt_c, allfull = st
            t_v = jnp.broadcast_to(t, (LANES,))
            m_v = jnp.broadcast_to(m, (LANES,))
            cnt_v = jnp.broadcast_to(cnt_c, (LANES,))
            cchunks = (cnt_c + LANES - 1) // LANES

            def emit_scatter(_):
                def sb(i, eqrun):
                    idx = cand_i[pl.ds(i * LANES, LANES)]
                    valid = (i * LANES + lane_iota) < cnt_v
                    u = cand_u[pl.ds(i * LANES, LANES)]
                    xg = _from_ord(u)
                    gt = jnp.logical_and(u > t_v, valid)
                    eq = jnp.logical_and(u == t_v, valid)
                    cs = jnp.cumsum(eq.astype(jnp.int32))
                    keep = jnp.logical_or(
                        gt, jnp.logical_and(eq, (cs + eqrun) <= m_v))
                    plsc.store_scatter(out_v, [idx], xg, mask=keep)
                    return eqrun + jnp.max(cs)
                lax.fori_loop(0, cchunks, sb, jnp.int32(0))
                return 0

            def emit_scan(_):
                def ob(i, eqrun):
                    xv = row_v[pl.ds(i * LANES, LANES)]
                    u = _to_ord(xv)
                    gt = u > t_v
                    eq = u == t_v
                    cs = jnp.cumsum(eq.astype(jnp.int32))
                    keep = jnp.logical_or(
                        gt, jnp.logical_and(eq, (cs + eqrun) <= m_v))
                    out_v[pl.ds(i * LANES, LANES)] = jnp.where(
                        keep, xv, zeros_f)
                    return eqrun + jnp.max(cs)
                lax.fori_loop(0, nchunks, ob, jnp.int32(0))
                return 0

            lax.cond(allfull, emit_scan, emit_scatter, 0)

        def restore_row(cand_i, st):
            _, _, cnt_c, allfull = st
            cnt_v = jnp.broadcast_to(cnt_c, (LANES,))
            cchunks = (cnt_c + LANES - 1) // LANES

            def restore_scatter(_):
                def rb(i, _):
                    idx = cand_i[pl.ds(i * LANES, LANES)]
                    valid = (i * LANES + lane_iota) < cnt_v
                    plsc.store_scatter(out_v, [idx], zeros_f, mask=valid)
                    return 0
                lax.fori_loop(0, cchunks, rb, 0)
                return 0

            def restore_all(_):
                def zb(i, _):
                    out_v[pl.ds(i * LANES, LANES)] = zeros_f
                    return 0
                lax.fori_loop(0, nchunks, zb, 0)
                return 0

            lax.cond(allfull, restore_all, restore_scatter, 0)

        r0 = wid * rows_per_worker
        cp_a = pltpu.async_copy(x_hbm.at[r0], row_a, sem_a)
        cp_b = pltpu.async_copy(x_hbm.at[r0 + 1], row_b, sem_b)

        def zout(i, _):
            out_v[pl.ds(i * LANES, LANES)] = zeros_f
            return 0
        lax.fori_loop(0, nchunks, zout, 0)

        def zhist(i, _):
            hist_v[pl.ds(i * LANES, LANES)] = zeros_i
            return 0
        lax.fori_loop(0, (LANES * HIST8) // LANES, zhist, 0)

        def zcand(i, _):
            cand_a[pl.ds(i * LANES, LANES)] = zeros_i
            cand_b[pl.ds(i * LANES, LANES)] = zeros_i
            return 0
        lax.fori_loop(0, (CAP + 144) // LANES, zcand, 0)

        cp_a.wait()
        st_a = select_row(row_a, cand_a)
        emit_row(row_a, cand_a, st_a)
        out_dma = pltpu.async_copy(out_v, out_hbm.at[r0], sem_out)
        cp_b.wait()
        st_b = select_row(row_b, cand_b)
        out_dma.wait()
        restore_row(cand_a, st_a)
        emit_row(row_b, cand_b, st_b)
        pltpu.sync_copy(out_v, out_hbm.at[r0 + 1])

    return topk_kernel


@jax.jit
def kernel(x):
    nrows, ncols = x.shape
    return _make_topk_kernel(nrows, ncols)(x)


# speedup vs baseline: 1.3470x; 1.0836x over previous
import functools

import jax
import jax.numpy as jnp
from jax import lax
from jax.experimental import pallas as pl
from jax.experimental.pallas import tpu as pltpu
from jax.experimental.pallas import tpu_sc as plsc

TOPK = 32
LANES = 16
NUM_CORES = 2
NUM_SUBCORES = 16
NUM_WORKERS = NUM_CORES * NUM_SUBCORES
HIST8 = 256
HIST4 = 16
CAP = 4096


def _to_ord(xv):
    b = lax.bitcast_convert_type(xv, jnp.int32)
    flip = (b >> 31) | jnp.int32(-2147483648)
    return lax.bitcast_convert_type(b ^ flip, jnp.uint32)


def _from_ord(u):
    ui = lax.bitcast_convert_type(u, jnp.int32)
    flip = ((~ui) >> 31) | jnp.int32(-2147483648)
    return lax.bitcast_convert_type(ui ^ flip, jnp.float32)


def _popcount_bytes(pm):
    b0 = (pm >> jnp.uint32(24)) & jnp.uint32(1)
    b1 = (pm >> jnp.uint32(16)) & jnp.uint32(1)
    b2 = (pm >> jnp.uint32(8)) & jnp.uint32(1)
    b3 = pm & jnp.uint32(1)
    return b0 + b1 + b2 + b3


def _make_topk_kernel(nrows, ncols):
    rows_per_worker = nrows // NUM_WORKERS
    assert rows_per_worker == 2 and ncols % (8 * LANES) == 0
    nchunks = ncols // LANES
    mesh = plsc.VectorSubcoreMesh(core_axis_name="c", subcore_axis_name="s")

    @functools.partial(
        pl.kernel,
        mesh=mesh,
        compiler_params=pltpu.CompilerParams(needs_layout_passes=False),
        out_type=jax.ShapeDtypeStruct((nrows, ncols), jnp.float32),
        scratch_types=[
            pltpu.VMEM((ncols,), jnp.float32),
            pltpu.VMEM((ncols,), jnp.float32),
            pltpu.VMEM((ncols,), jnp.float32),
            pltpu.VMEM((LANES * HIST8,), jnp.int32),
            pltpu.VMEM((CAP + 144,), jnp.int32),
            pltpu.VMEM((CAP + 144,), jnp.int32),
            pltpu.VMEM((CAP + 144,), jnp.uint32),
            pltpu.SemaphoreType.DMA,
            pltpu.SemaphoreType.DMA,
            pltpu.SemaphoreType.DMA,
        ],
    )
    def topk_kernel(x_hbm, out_hbm, row_a, row_b, out_v, hist_v, cand_a,
                    cand_b, cand_u, sem_a, sem_b, sem_out):
        wid = lax.axis_index("s") * NUM_CORES + lax.axis_index("c")
        lane_iota = lax.iota(jnp.int32, LANES)
        ones_i = jnp.ones((LANES,), jnp.int32)
        zeros_i = jnp.zeros((LANES,), jnp.int32)
        zeros_f = jnp.zeros((LANES,), jnp.float32)

        def top2_scan(row_v):
            zf = jnp.full((LANES,), -jnp.inf, jnp.float32)

            def body(o, carry):
                m1s = list(carry[:8])
                m2s = list(carry[8:])
                for s in range(8):
                    xv = row_v[pl.ds((o * 8 + s) * LANES, LANES)]
                    m2s[s] = jnp.maximum(m2s[s], jnp.minimum(m1s[s], xv))
                    m1s[s] = jnp.maximum(m1s[s], xv)
                return tuple(m1s + m2s)
            carry = lax.fori_loop(0, nchunks // 8, body, (zf,) * 16)

            pairs = list(zip(carry[:8], carry[8:]))
            while len(pairs) > 1:
                nxt = []
                for (a1, a2), (b1, b2) in zip(pairs[::2], pairs[1::2]):
                    hi = jnp.maximum(a1, b1)
                    lo = jnp.maximum(jnp.minimum(a1, b1),
                                     jnp.maximum(a2, b2))
                    nxt.append((hi, lo))
                pairs = nxt
            _, m2 = pairs[0]
            return jnp.min(m2)

        def compact(row_v, cand_i, thresh, in_ord):
            th_v = jnp.broadcast_to(thresh, (LANES,))

            def cb(o, ptr):
                base = jnp.minimum(ptr, jnp.int32(CAP))
                masks, cnts = [], []
                for s in range(8):
                    xv = row_v[pl.ds((o * 8 + s) * LANES, LANES)]
                    matc = (_to_ord(xv) if in_ord else xv) >= th_v
                    masks.append(matc)
                    cnts.append(jnp.sum(matc.astype(jnp.int32)))
                offs = [base]
                for s in range(8):
                    offs.append(offs[s] + cnts[s])
                for s in range(8):
                    plsc.store_compressed(
                        cand_i.at[pl.ds(offs[s], LANES)],
                        lane_iota + (o * 8 + s) * LANES, mask=masks[s])
                return ptr + (offs[8] - base)
            return lax.fori_loop(0, nchunks // 8, cb, jnp.int32(0))

        def find_top(krem, nbuckets):
            def wcond(carry):
                _, cum, _ = carry
                return cum < krem

            def wbody(carry):
                c, cum, _ = carry
                v = hist_v[pl.ds(c * LANES, LANES)]
                hist_v[pl.ds(c * LANES, LANES)] = zeros_i
                return c - 1, cum + jnp.sum(v), cum
            c, cum, prev = lax.while_loop(
                wcond, wbody,
                (jnp.int32(nbuckets - 1), jnp.int32(0), jnp.int32(0)))
            d = c + 1

            def zb(b, _):
                hist_v[pl.ds(b * LANES, LANES)] = zeros_i
                return 0
            lax.fori_loop(0, d, zb, 0)
            return d, prev, cum - prev

        def full_round(row_v, shift, pm, pb, krem, masked):
            def hb(o, _):
                for s in range(8):
                    u = _to_ord(row_v[pl.ds((o * 8 + s) * LANES, LANES)])
                    digit = ((u >> shift) & jnp.uint32(0xFF)).astype(jnp.int32)
                    idx = digit * LANES + lane_iota
                    if masked:
                        matc = (u & pm) == pb
                        plsc.addupdate_scatter(hist_v, [idx], ones_i,
                                               mask=matc)
                    else:
                        plsc.addupdate_scatter(hist_v, [idx], ones_i)
                return 0
            lax.fori_loop(0, nchunks // 8, hb, 0)

            d, ca, ceq = find_top(krem, HIST8)
            pb = pb | (d.astype(jnp.uint32) << shift)
            pm = pm | (jnp.uint32(0xFF) << shift)
            krem = krem - ca
            return pm, pb, krem, ceq

        def select_row(row_v, cand_i):
            T = top2_scan(row_v)
            c_t = compact(row_v, cand_i, T, in_ord=False)

            def fast(_):
                return (jnp.uint32(0), jnp.uint32(0), jnp.int32(TOPK), c_t,
                        jnp.int32(8))

            def slow(_):
                pm, pb, krem, ceq = full_round(
                    row_v, jnp.uint32(24), jnp.uint32(0), jnp.uint32(0),
                    jnp.int32(TOPK), masked=False)

                def esc_body(rnd, carry):
                    pm, pb, krem, ceq = carry

                    def run(_):
                        shift = (jnp.uint32(24)
                                 - jnp.uint32(8) * rnd.astype(jnp.uint32))
                        return full_round(row_v, shift, pm, pb, krem,
                                          masked=True)
                    return lax.cond(
                        (TOPK - krem) + ceq > CAP, run,
                        lambda _: (pm, pb, krem, ceq), 0)
                pm, pb, krem, ceq = lax.fori_loop(
                    1, 4, esc_body, (pm, pb, krem, ceq))

                compact(row_v, cand_i, pb, in_ord=True)
                cnt = (TOPK - krem) + ceq
                nrounds = ((jnp.uint32(4) - _popcount_bytes(pm))
                           * jnp.uint32(2)).astype(jnp.int32)
                return pm, pb, krem, cnt, nrounds

            pm, pb, krem, cnt_c, nrounds = lax.cond(
                c_t <= CAP, fast, slow, 0)
            cchunks = (cnt_c + LANES - 1) // LANES
            cnt_v = jnp.broadcast_to(cnt_c, (LANES,))
            nbits = jnp.uint32(4) * nrounds.astype(jnp.uint32)

            def gb(j, _):
                idx = cand_i[pl.ds(j * LANES, LANES)]
                valid = (j * LANES + lane_iota) < cnt_v
                xg = plsc.load_gather(row_v, [idx], mask=valid)
                cand_u[pl.ds(j * LANES, LANES)] = _to_ord(xg)
                return 0
            lax.fori_loop(0, cchunks, gb, 0)

            def cr_body(i, carry):
                pm2, pb2, krem2 = carry
                shift = nbits - jnp.uint32(4) * (i.astype(jnp.uint32)
                                                 + jnp.uint32(1))

                def chb(j, _):
                    u = cand_u[pl.ds(j * LANES, LANES)]
                    valid = (j * LANES + lane_iota) < cnt_v
                    matc = jnp.logical_and(valid, (u & pm2) == pb2)
                    digit = ((u >> shift) & jnp.uint32(0xF)).astype(jnp.int32)
                    plsc.addupdate_scatter(
                        hist_v, [digit * LANES + lane_iota], ones_i,
                        mask=matc)
                    return 0
                lax.fori_loop(0, cchunks, chb, 0)

                d, excl, _ = find_top(krem2, HIST4)
                pb2 = pb2 | (d.astype(jnp.uint32) << shift)
                pm2 = pm2 | (jnp.uint32(0xF) << shift)
                krem2 = krem2 - excl
                return pm2, pb2, krem2

            _, t, m = lax.fori_loop(0, nrounds, cr_body, (pm, pb, krem))
            return t, m, cnt_c, nrounds == 0

        def emit_row(row_v, cand_i, st):
            t, m, cnt_c, allfull = st
            t_v = jnp.broadcast_to(t, (LANES,))
            m_v = jnp.broadcast_to(m, (LANES,))
            cnt_v = jnp.broadcast_to(cnt_c, (LANES,))
            cchunks = (cnt_c + LANES - 1) // LANES

            def emit_scatter(_):
                def sb(i, eqrun):
                    idx = cand_i[pl.ds(i * LANES, LANES)]
                    valid = (i * LANES + lane_iota) < cnt_v
                    u = cand_u[pl.ds(i * LANES, LANES)]
                    xg = _from_ord(u)
                    gt = jnp.logical_and(u > t_v, valid)
                    eq = jnp.logical_and(u == t_v, valid)
                    cs = jnp.cumsum(eq.astype(jnp.int32))
                    keep = jnp.logical_or(
                        gt, jnp.logical_and(eq, (cs + eqrun) <= m_v))
                    plsc.store_scatter(out_v, [idx], xg, mask=keep)
                    return eqrun + jnp.max(cs)
                lax.fori_loop(0, cchunks, sb, jnp.int32(0))
                return 0

            def emit_scan(_):
                def ob(i, eqrun):
                    xv = row_v[pl.ds(i * LANES, LANES)]
                    u = _to_ord(xv)
                    gt = u > t_v
                    eq = u == t_v
                    cs = jnp.cumsum(eq.astype(jnp.int32))
                    keep = jnp.logical_or(
                        gt, jnp.logical_and(eq, (cs + eqrun) <= m_v))
                    out_v[pl.ds(i * LANES, LANES)] = jnp.where(
                        keep, xv, zeros_f)
                    return eqrun + jnp.max(cs)
                lax.fori_loop(0, nchunks, ob, jnp.int32(0))
                return 0

            lax.cond(allfull, emit_scan, emit_scatter, 0)

        def restore_row(cand_i, st):
            _, _, cnt_c, allfull = st
            cnt_v = jnp.broadcast_to(cnt_c, (LANES,))
            cchunks = (cnt_c + LANES - 1) // LANES

            def restore_scatter(_):
                def rb(i, _):
                    idx = cand_i[pl.ds(i * LANES, LANES)]
                    valid = (i * LANES + lane_iota) < cnt_v
                    plsc.store_scatter(out_v, [idx], zeros_f, mask=valid)
                    return 0
                lax.fori_loop(0, cchunks, rb, 0)
                return 0

            def restore_all(_):
                def zb(i, _):
                    out_v[pl.ds(i * LANES, LANES)] = zeros_f
                    return 0
                lax.fori_loop(0, nchunks, zb, 0)
                return 0

            lax.cond(allfull, restore_all, restore_scatter, 0)

        r0 = wid * rows_per_worker
        cp_a = pltpu.async_copy(x_hbm.at[r0], row_a, sem_a)
        cp_b = pltpu.async_copy(x_hbm.at[r0 + 1], row_b, sem_b)

        def zout(i, _):
            out_v[pl.ds(i * LANES, LANES)] = zeros_f
            return 0
        lax.fori_loop(0, nchunks, zout, 0)

        def zhist(i, _):
            hist_v[pl.ds(i * LANES, LANES)] = zeros_i
            return 0
        lax.fori_loop(0, (LANES * HIST8) // LANES, zhist, 0)

        def zcand(i, _):
            cand_a[pl.ds(i * LANES, LANES)] = zeros_i
            cand_b[pl.ds(i * LANES, LANES)] = zeros_i
            return 0
        lax.fori_loop(0, (CAP + 144) // LANES, zcand, 0)

        cp_a.wait()
        st_a = select_row(row_a, cand_a)
        emit_row(row_a, cand_a, st_a)
        out_dma = pltpu.async_copy(out_v, out_hbm.at[r0], sem_out)
        cp_b.wait()
        st_b = select_row(row_b, cand_b)
        out_dma.wait()
        restore_row(cand_a, st_a)
        emit_row(row_b, cand_b, st_b)
        pltpu.sync_copy(out_v, out_hbm.at[r0 + 1])

    return topk_kernel


@jax.jit
def kernel(x):
    nrows, ncols = x.shape
    return _make_topk_kernel(nrows, ncols)(x)
